# Initial kernel scaffold; baseline (speedup 1.0000x reference)
import jax
import jax.numpy as jnp
from jax.experimental import pallas as pl


def kernel(src_seq, emb_table, W1, b1, W2, b2):
    raise NotImplementedError("write your pallas kernel here")


# speedup vs baseline: 2.1684x; 2.1684x over previous
import functools

import jax
import jax.numpy as jnp
from jax import lax
from jax.experimental import pallas as pl
from jax.experimental.pallas import tpu as pltpu
from jax.experimental.pallas import tpu_sc as plsc

_B = 4096
_L = 50
_HIDDEN = 128
_CODE = 128
_R = _B * _L

_NC = 2
_NS = 16
_NW = _NC * _NS
_R_PER_W = _R // _NW
_CHUNK = 400
_NCHUNK = _R_PER_W // _CHUNK


def _sc_gather(src_idx, emb_table):
    mesh = plsc.VectorSubcoreMesh(core_axis_name="c", subcore_axis_name="s")

    @functools.partial(
        pl.kernel,
        mesh=mesh,
        out_type=jax.ShapeDtypeStruct((_R, _HIDDEN), jnp.float32),
        scratch_types=[
            pltpu.VMEM((_R_PER_W,), jnp.int32),
            pltpu.VMEM((_CHUNK, _HIDDEN), jnp.float32),
            pltpu.VMEM((_CHUNK, _HIDDEN), jnp.float32),
            pltpu.SemaphoreType.DMA,
            pltpu.SemaphoreType.DMA,
        ],
    )
    def gather_kernel(idx_hbm, table_hbm, out_hbm, idx_v, rows0, rows1, sem0, sem1):
        wid = lax.axis_index("s") * _NC + lax.axis_index("c")
        base = wid * _R_PER_W
        pltpu.sync_copy(idx_hbm.at[pl.ds(base, _R_PER_W)], idx_v)

        bufs = (rows0, rows1)
        sems = (sem0, sem1)

        def start(c, slot):
            pltpu.async_copy(
                table_hbm.at[idx_v.at[pl.ds(c * _CHUNK, _CHUNK)]],
                bufs[slot],
                sems[slot],
            )

        def finish(c, slot):
            pltpu.make_async_copy(
                table_hbm.at[idx_v.at[pl.ds(c * _CHUNK, _CHUNK)]],
                bufs[slot],
                sems[slot],
            ).wait()
            pltpu.sync_copy(bufs[slot], out_hbm.at[pl.ds(base + c * _CHUNK, _CHUNK)])

        start(0, 0)
        start(1, 1)

        def body(i, _):
            c0 = i * 2
            for b in range(2):
                c = c0 + b
                finish(c, b)

                @pl.when(c + 2 < _NCHUNK)
                def _():
                    start(c + 2, b)

            return 0

        lax.fori_loop(0, _NCHUNK // 2, body, 0)

    return gather_kernel(src_idx, emb_table)


_BLK = 2048


def _mlp_body(x_ref, w1_ref, b1_ref, w2_ref, b2_ref, o_ref):
    x = x_ref[...]
    h = jnp.dot(x, w1_ref[...], preferred_element_type=jnp.float32)
    h = jnp.maximum(h + b1_ref[...], 0.0)
    o = jnp.dot(h, w2_ref[...], preferred_element_type=jnp.float32)
    o_ref[...] = jnp.maximum(o + b2_ref[...], 0.0)


def _tc_mlp(enc, W1, b1, W2, b2):
    return pl.pallas_call(
        _mlp_body,
        grid=(_R // _BLK,),
        in_specs=[
            pl.BlockSpec((_BLK, _HIDDEN), lambda i: (i, 0)),
            pl.BlockSpec((_HIDDEN, _HIDDEN), lambda i: (0, 0)),
            pl.BlockSpec((1, _HIDDEN), lambda i: (0, 0)),
            pl.BlockSpec((_HIDDEN, _CODE), lambda i: (0, 0)),
            pl.BlockSpec((1, _CODE), lambda i: (0, 0)),
        ],
        out_specs=pl.BlockSpec((_BLK, _CODE), lambda i: (i, 0)),
        out_shape=jax.ShapeDtypeStruct((_R, _CODE), jnp.float32),
    )(enc, W1, b1[None, :], W2, b2[None, :])


def kernel(src_seq, emb_table, W1, b1, W2, b2):
    idx = src_seq.reshape(_R).astype(jnp.int32)
    enc = _sc_gather(idx, emb_table)
    out = _tc_mlp(enc, W1, b1, W2, b2)
    return out.reshape(_B, _L, _CODE)


# speedup vs baseline: 2.1731x; 1.0022x over previous
import functools

import jax
import jax.numpy as jnp
from jax import lax
from jax.experimental import pallas as pl
from jax.experimental.pallas import tpu as pltpu
from jax.experimental.pallas import tpu_sc as plsc

_B = 4096
_L = 50
_HIDDEN = 128
_CODE = 128
_R = _B * _L

_NC = 2
_NS = 16
_NW = _NC * _NS
_R_PER_W = _R // _NW
_CHUNK = 400
_NCHUNK = _R_PER_W // _CHUNK


def _sc_gather(src_idx, emb_table):
    mesh = plsc.VectorSubcoreMesh(core_axis_name="c", subcore_axis_name="s")

    @functools.partial(
        pl.kernel,
        mesh=mesh,
        out_type=jax.ShapeDtypeStruct((_R, _HIDDEN), jnp.float32),
        compiler_params=pltpu.CompilerParams(use_tc_tiling_on_sc=True),
        scratch_types=[
            pltpu.VMEM((_R_PER_W,), jnp.int32),
            pltpu.VMEM((_CHUNK, _HIDDEN), jnp.float32),
            pltpu.VMEM((_CHUNK, _HIDDEN), jnp.float32),
            pltpu.SemaphoreType.DMA,
            pltpu.SemaphoreType.DMA,
        ],
    )
    def gather_kernel(idx_hbm, table_hbm, out_hbm, idx_v, rows0, rows1, sem0, sem1):
        wid = lax.axis_index("s") * _NC + lax.axis_index("c")
        base = wid * _R_PER_W
        pltpu.sync_copy(idx_hbm.at[pl.ds(base, _R_PER_W)], idx_v)

        bufs = (rows0, rows1)
        sems = (sem0, sem1)

        def start(c, slot):
            pltpu.async_copy(
                table_hbm.at[idx_v.at[pl.ds(c * _CHUNK, _CHUNK)]],
                bufs[slot],
                sems[slot],
            )

        def finish(c, slot):
            pltpu.make_async_copy(
                table_hbm.at[idx_v.at[pl.ds(c * _CHUNK, _CHUNK)]],
                bufs[slot],
                sems[slot],
            ).wait()
            pltpu.sync_copy(bufs[slot], out_hbm.at[pl.ds(base + c * _CHUNK, _CHUNK)])

        start(0, 0)
        start(1, 1)

        def body(i, _):
            c0 = i * 2
            for b in range(2):
                c = c0 + b
                finish(c, b)

                @pl.when(c + 2 < _NCHUNK)
                def _():
                    start(c + 2, b)

            return 0

        lax.fori_loop(0, _NCHUNK // 2, body, 0)

    return gather_kernel(src_idx, emb_table)


_BLK = 2048


def _mlp_body(x_ref, w1_ref, b1_ref, w2_ref, b2_ref, o_ref):
    x = x_ref[...]
    h = jnp.dot(x, w1_ref[...], preferred_element_type=jnp.float32)
    h = jnp.maximum(h + b1_ref[...], 0.0)
    o = jnp.dot(h, w2_ref[...], preferred_element_type=jnp.float32)
    o_ref[...] = jnp.maximum(o + b2_ref[...], 0.0)


def _tc_mlp(enc, W1, b1, W2, b2):
    return pl.pallas_call(
        _mlp_body,
        grid=(_R // _BLK,),
        in_specs=[
            pl.BlockSpec((_BLK, _HIDDEN), lambda i: (i, 0)),
            pl.BlockSpec((_HIDDEN, _HIDDEN), lambda i: (0, 0)),
            pl.BlockSpec((1, _HIDDEN), lambda i: (0, 0)),
            pl.BlockSpec((_HIDDEN, _CODE), lambda i: (0, 0)),
            pl.BlockSpec((1, _CODE), lambda i: (0, 0)),
        ],
        out_specs=pl.BlockSpec((_BLK, _CODE), lambda i: (i, 0)),
        out_shape=jax.ShapeDtypeStruct((_R, _CODE), jnp.float32),
    )(enc, W1, b1[None, :], W2, b2[None, :])


def kernel(src_seq, emb_table, W1, b1, W2, b2):
    idx = src_seq.reshape(_R).astype(jnp.int32)
    enc = _sc_gather(idx, emb_table)
    out = _tc_mlp(enc, W1, b1, W2, b2)
    return out.reshape(_B, _L, _CODE)


# speedup vs baseline: 4.1917x; 1.9289x over previous
import functools

import jax
import jax.numpy as jnp
from jax import lax
from jax.experimental import pallas as pl
from jax.experimental.pallas import tpu as pltpu
from jax.experimental.pallas import tpu_sc as plsc

_B = 4096
_L = 50
_HIDDEN = 128
_CODE = 128
_R = _B * _L

_NC = 2
_NS = 16
_NW = _NC * _NS
_R_PER_W = _R // _NW
_CHUNK = 400
_NCHUNK = _R_PER_W // _CHUNK


def _sc_gather(src_idx, emb_table):
    mesh = plsc.VectorSubcoreMesh(core_axis_name="c", subcore_axis_name="s")

    @functools.partial(
        pl.kernel,
        mesh=mesh,
        out_type=jax.ShapeDtypeStruct((_R, _HIDDEN), jnp.float32),
        compiler_params=pltpu.CompilerParams(use_tc_tiling_on_sc=True),
        scratch_types=[
            pltpu.VMEM((_R_PER_W,), jnp.int32),
            pltpu.VMEM((_CHUNK, _HIDDEN), jnp.float32),
            pltpu.VMEM((_CHUNK, _HIDDEN), jnp.float32),
            pltpu.SemaphoreType.DMA,
            pltpu.SemaphoreType.DMA,
        ],
    )
    def gather_kernel(idx_hbm, table_hbm, out_hbm, idx_v, rows0, rows1, sem0, sem1):
        wid = lax.axis_index("s") * _NC + lax.axis_index("c")
        base = wid * _R_PER_W
        pltpu.sync_copy(idx_hbm.at[pl.ds(base, _R_PER_W)], idx_v)

        bufs = (rows0, rows1)
        sems = (sem0, sem1)

        def start(c, slot):
            pltpu.async_copy(
                table_hbm.at[idx_v.at[pl.ds(c * _CHUNK, _CHUNK)]],
                bufs[slot],
                sems[slot],
            )

        def finish(c, slot):
            pltpu.make_async_copy(
                table_hbm.at[idx_v.at[pl.ds(c * _CHUNK, _CHUNK)]],
                bufs[slot],
                sems[slot],
            ).wait()
            pltpu.sync_copy(bufs[slot], out_hbm.at[pl.ds(base + c * _CHUNK, _CHUNK)])

        start(0, 0)
        start(1, 1)

        def body(i, _):
            c0 = i * 2
            for b in range(2):
                c = c0 + b
                finish(c, b)

                @pl.when(c + 2 < _NCHUNK)
                def _():
                    start(c + 2, b)

            return 0

        lax.fori_loop(0, _NCHUNK // 2, body, 0)

    return gather_kernel(src_idx, emb_table)


_BLK = 2048


def _mlp_body(x_ref, w1_ref, b1_ref, w2_ref, b2_ref, o_ref):
    x = x_ref[...]
    h = jnp.dot(x, w1_ref[...], preferred_element_type=jnp.float32)
    h = jnp.maximum(h + b1_ref[...], 0.0)
    o = jnp.dot(h, w2_ref[...], preferred_element_type=jnp.float32)
    o_ref[...] = jnp.maximum(o + b2_ref[...], 0.0)


def _tc_mlp(enc, W1, b1, W2, b2):
    return pl.pallas_call(
        _mlp_body,
        grid=(_R // _BLK,),
        in_specs=[
            pl.BlockSpec((_BLK, _HIDDEN), lambda i: (i, 0)),
            pl.BlockSpec((_HIDDEN, _HIDDEN), lambda i: (0, 0)),
            pl.BlockSpec((1, _HIDDEN), lambda i: (0, 0)),
            pl.BlockSpec((_HIDDEN, _CODE), lambda i: (0, 0)),
            pl.BlockSpec((1, _CODE), lambda i: (0, 0)),
        ],
        out_specs=pl.BlockSpec((_BLK, _CODE), lambda i: (i, 0)),
        out_shape=jax.ShapeDtypeStruct((_R, _CODE), jnp.float32),
    )(enc, W1, b1[None, :], W2, b2[None, :])


def kernel(src_seq, emb_table, W1, b1, W2, b2):
    idx = src_seq.T.reshape(_R).astype(jnp.int32)
    enc = _sc_gather(idx, emb_table)
    out = _tc_mlp(enc, W1, b1, W2, b2)
    return out.reshape(_L, _B, _CODE).transpose(1, 0, 2)


# speedup vs baseline: 4.1983x; 1.0016x over previous
import functools

import jax
import jax.numpy as jnp
from jax import lax
from jax.experimental import pallas as pl
from jax.experimental.pallas import tpu as pltpu
from jax.experimental.pallas import tpu_sc as plsc

_B = 4096
_L = 50
_HIDDEN = 128
_CODE = 128
_R = _B * _L

_NC = 2
_NS = 16
_NW = _NC * _NS
_R_PER_W = _R // _NW
_CHUNK = 400
_NCHUNK = _R_PER_W // _CHUNK


def _sc_gather(src_idx, emb_table):
    mesh = plsc.VectorSubcoreMesh(core_axis_name="c", subcore_axis_name="s")

    @functools.partial(
        pl.kernel,
        mesh=mesh,
        out_type=jax.ShapeDtypeStruct((_R, _HIDDEN), jnp.float32),
        compiler_params=pltpu.CompilerParams(use_tc_tiling_on_sc=True),
        scratch_types=[
            pltpu.VMEM((_R_PER_W,), jnp.int32),
            pltpu.VMEM((_CHUNK, _HIDDEN), jnp.float32),
            pltpu.VMEM((_CHUNK, _HIDDEN), jnp.float32),
            pltpu.SemaphoreType.DMA,
            pltpu.SemaphoreType.DMA,
        ],
    )
    def gather_kernel(idx_hbm, table_hbm, out_hbm, idx_v, rows0, rows1, sem0, sem1):
        wid = lax.axis_index("s") * _NC + lax.axis_index("c")
        base = wid * _R_PER_W
        pltpu.sync_copy(idx_hbm.at[pl.ds(base, _R_PER_W)], idx_v)

        bufs = (rows0, rows1)
        sems = (sem0, sem1)

        def start(c, slot):
            pltpu.async_copy(
                table_hbm.at[idx_v.at[pl.ds(c * _CHUNK, _CHUNK)]],
                bufs[slot],
                sems[slot],
            )

        def finish(c, slot):
            pltpu.make_async_copy(
                table_hbm.at[idx_v.at[pl.ds(c * _CHUNK, _CHUNK)]],
                bufs[slot],
                sems[slot],
            ).wait()
            pltpu.sync_copy(bufs[slot], out_hbm.at[pl.ds(base + c * _CHUNK, _CHUNK)])

        start(0, 0)
        start(1, 1)

        def body(i, _):
            c0 = i * 2
            for b in range(2):
                c = c0 + b
                finish(c, b)

                @pl.when(c + 2 < _NCHUNK)
                def _():
                    start(c + 2, b)

            return 0

        lax.fori_loop(0, _NCHUNK // 2, body, 0)

    return gather_kernel(src_idx, emb_table)


_BLK = 2048


def _mlp_body(x_ref, w1_ref, b1_ref, w2_ref, b2_ref, o_ref):
    x = x_ref[...].astype(jnp.bfloat16)
    w1 = w1_ref[...].astype(jnp.bfloat16)
    w2 = w2_ref[...].astype(jnp.bfloat16)
    h = jnp.dot(x, w1, preferred_element_type=jnp.float32)
    h = jnp.maximum(h + b1_ref[...], 0.0).astype(jnp.bfloat16)
    o = jnp.dot(h, w2, preferred_element_type=jnp.float32)
    o_ref[...] = jnp.maximum(o + b2_ref[...], 0.0)


def _tc_mlp(enc, W1, b1, W2, b2):
    return pl.pallas_call(
        _mlp_body,
        grid=(_R // _BLK,),
        in_specs=[
            pl.BlockSpec((_BLK, _HIDDEN), lambda i: (i, 0)),
            pl.BlockSpec((_HIDDEN, _HIDDEN), lambda i: (0, 0)),
            pl.BlockSpec((1, _HIDDEN), lambda i: (0, 0)),
            pl.BlockSpec((_HIDDEN, _CODE), lambda i: (0, 0)),
            pl.BlockSpec((1, _CODE), lambda i: (0, 0)),
        ],
        out_specs=pl.BlockSpec((_BLK, _CODE), lambda i: (i, 0)),
        out_shape=jax.ShapeDtypeStruct((_R, _CODE), jnp.float32),
    )(enc, W1, b1[None, :], W2, b2[None, :])


def kernel(src_seq, emb_table, W1, b1, W2, b2):
    idx = src_seq.T.reshape(_R).astype(jnp.int32)
    enc = _sc_gather(idx, emb_table)
    out = _tc_mlp(enc, W1, b1, W2, b2)
    return out.reshape(_L, _B, _CODE).transpose(1, 0, 2)
